# baseline (speedup 1.0000x reference)
import functools

import jax
import jax.numpy as jnp
from jax import lax
from jax.experimental import pallas as pl
from jax.experimental.pallas import tpu as pltpu
from jax.experimental.pallas import tpu_sc as plsc

N_USER = 25000
N_ITEM = 25000
NNODE = N_USER + N_ITEM
D = 64
DH = 32
L = 3
E = 800000
B = 4096

NC, NS = 2, 16
K = 256
NSUB = K // 128
NBLK = 196
ET = NBLK * K
EP = NS * ET
NPAD = 50176
RPT = NPAD // NS
P4 = NPAD // 4
P2 = NPAD // 2
RP = 1792
GRID = P4 // RP

_mesh = plsc.VectorSubcoreMesh(core_axis_name="c", subcore_axis_name="s")
_sc_params = pltpu.CompilerParams(use_tc_tiling_on_sc=False)


@functools.partial(
    pl.kernel,
    out_type=jax.ShapeDtypeStruct((NC, NPAD, DH), jnp.float32),
    mesh=_mesh,
    scratch_types=[
        pltpu.MemorySpace.VMEM_SHARED((NPAD, DH), jnp.float32),
        pltpu.MemorySpace.VMEM((4, NSUB, 128), jnp.int32),
        pltpu.MemorySpace.VMEM((4, NSUB, 128), jnp.int32),
        pltpu.MemorySpace.VMEM((4, K), jnp.float32),
        pltpu.MemorySpace.VMEM((2, NSUB, 128), jnp.int32),
        pltpu.MemorySpace.VMEM((2, K, DH), jnp.float32),
        pltpu.SemaphoreType.DMA((4,)),
        pltpu.SemaphoreType.DMA((2,)),
        pltpu.SemaphoreType.DMA((2,)),
    ],
    compiler_params=_sc_params,
)
def _spmm(tbl_hbm, rc_hbm, val_hbm, out_hbm,
          acc, colb, rowb, valb, rowS, gath, isem, gsem, ssem):
    c = lax.axis_index("c")
    t = lax.axis_index("s")

    def zrow(i, _):
        gath[0, i, pl.ds(0, 16)] = jnp.zeros((16,), jnp.float32)
        gath[0, i, pl.ds(16, 16)] = jnp.zeros((16,), jnp.float32)
        return 0
    lax.fori_loop(0, K, zrow, 0)
    zd = []
    for off in range(0, RPT, K):
        sz = min(K, RPT - off)
        zd.append(pltpu.async_copy(gath.at[0, pl.ds(0, sz)],
                                   acc.at[pl.ds(t * RPT + off, sz)],
                                   ssem.at[0]))
    for dd in zd:
        dd.wait()

    off_c = c

    def issue_idx(b, r):
        pltpu.async_copy(rc_hbm.at[1, t, b], colb.at[r], isem.at[r])
        pltpu.async_copy(rc_hbm.at[0, t, b], rowb.at[r], isem.at[r])
        pltpu.async_copy(val_hbm.at[t, b], valb.at[r], isem.at[r])

    def drain_idx(b, r):
        pltpu.make_async_copy(rc_hbm.at[1, t, b], colb.at[r],
                              isem.at[r]).wait()
        pltpu.make_async_copy(rc_hbm.at[0, t, b], rowb.at[r],
                              isem.at[r]).wait()
        pltpu.make_async_copy(val_hbm.at[t, b], valb.at[r],
                              isem.at[r]).wait()

    def drain_scat(s):
        for j in range(NSUB):
            pltpu.make_async_copy(gath.at[s, pl.ds(j * 128, 128)],
                                  acc.at[rowS.at[s, j]], ssem.at[s]).wait()

    def produce(b, r, s, first):
        drain_idx(b, r)

        def obody(ii, _):
            for j in range(NSUB):
                colb[r, j, pl.ds(ii * 16, 16)] = (
                    colb[r, j, pl.ds(ii * 16, 16)] * 2 + off_c)
            return 0
        lax.fori_loop(0, 8, obody, 0, unroll=True)

        def after_credit():
            for j in range(NSUB):
                def cbody(ii, _):
                    rowS[s, j, pl.ds(ii * 16, 16)] = rowb[r, j,
                                                          pl.ds(ii * 16, 16)]
                    return 0
                lax.fori_loop(0, 8, cbody, 0, unroll=True)

        if first:
            @pl.when(b >= 2)
            def _():
                drain_scat(s)
        else:
            drain_scat(s)
        after_credit()
        return [pltpu.async_copy(tbl_hbm.at[colb.at[r, j]],
                                 gath.at[s, pl.ds(j * 128, 128)], gsem.at[s])
                for j in range(NSUB)]

    def scale(s, r):
        @plsc.parallel_loop(0, K // 16, 1, unroll=4)
        def _(g):
            vv = valb[r, pl.ds(g * 16, 16)]
            base = g * 16
            for j in range(16):
                v = vv[j]
                gath[s, base + j, pl.ds(0, 16)] = (
                    gath[s, base + j, pl.ds(0, 16)] * v)
                gath[s, base + j, pl.ds(16, 16)] = (
                    gath[s, base + j, pl.ds(16, 16)] * v)

    def consume(x, r, s):
        pltpu.make_async_copy(tbl_hbm.at[colb.at[r, 0]],
                              gath.at[s, pl.ds(0, 128)], gsem.at[s]).wait()
        pltpu.make_async_copy(tbl_hbm.at[colb.at[r, 1]],
                              gath.at[s, pl.ds(128, 128)], gsem.at[s]).wait()
        scale(s, r)
        for j in range(NSUB):
            pltpu.async_copy(gath.at[s, pl.ds(j * 128, 128)],
                             acc.at[rowS.at[s, j]], ssem.at[s], add=True)

    for bb in range(3):
        issue_idx(bb, bb)
    plsc.subcore_barrier()

    NB = ET // K

    def quad_body(o, _):
        for u in range(4):
            b = 4 * o + u
            s = u % 2
            gd = produce(b, u, s, first=(u < 2))
            xu = (u - 1) % 4
            xs = (u - 1) % 2
            pref = b + 2
            if u == 0:
                @pl.when(o > 0)
                def _():
                    consume(b - 1, xu, xs)
                    issue_idx(pref, (u + 2) % 4)
            elif u == 1:
                consume(b - 1, xu, xs)
                issue_idx(pref, (u + 2) % 4)
            else:
                consume(b - 1, xu, xs)

                @pl.when(o < NB // 4 - 1)
                def _():
                    issue_idx(pref, (u + 2) % 4)
        return 0

    lax.fori_loop(0, NB // 4, quad_body, 0)
    consume(NB - 1, (NB - 1) % 4, (NB - 1) % 2)
    drain_scat(0)
    drain_scat(1)
    plsc.subcore_barrier()
    pltpu.sync_copy(acc.at[pl.ds(t * RPT, RPT)],
                    out_hbm.at[c, pl.ds(t * RPT, RPT)])


def _tc_transform_body(s_ref, e_ref, pa_ref, pb_ref, wg_ref, wb_ref,
                       b2_ref, gg_ref, ego_o_ref, norm_o_ref):
    f32 = jnp.float32
    snm = (jnp.dot(s_ref[0], pa_ref[...], preferred_element_type=f32)
           + jnp.dot(s_ref[1], pb_ref[...], preferred_element_type=f32)
           ).reshape(2 * RP, 128)
    e = e_ref[...]
    z = (jnp.dot(snm, wg_ref[...], preferred_element_type=f32)
         + jnp.dot(e * snm, wb_ref[...], preferred_element_type=f32)
         + b2_ref[...])
    y = jnp.where(z >= 0, z, 0.2 * z)
    nrmb = jnp.dot(y * y, gg_ref[...], preferred_element_type=f32)
    norm_o_ref[...] = y / jnp.maximum(jnp.sqrt(nrmb), 1e-12)
    ego_o_ref[...] = y


_tc_transform = pl.pallas_call(
    _tc_transform_body,
    grid=(GRID,),
    in_specs=[
        pl.BlockSpec((NC, RP, 128), lambda i: (0, i, 0)),
        pl.BlockSpec((2 * RP, 128), lambda i: (i, 0)),
        pl.BlockSpec((128, 256), lambda i: (0, 0)),
        pl.BlockSpec((128, 256), lambda i: (0, 0)),
        pl.BlockSpec((128, 128), lambda i: (0, 0)),
        pl.BlockSpec((128, 128), lambda i: (0, 0)),
        pl.BlockSpec((1, 128), lambda i: (0, 0)),
        pl.BlockSpec((128, 128), lambda i: (0, 0)),
    ],
    out_specs=[
        pl.BlockSpec((2 * RP, 128), lambda i: (i, 0)),
        pl.BlockSpec((2 * RP, 128), lambda i: (i, 0)),
    ],
    out_shape=[
        jax.ShapeDtypeStruct((P2, 128), jnp.float32),
        jax.ShapeDtypeStruct((P2, 128), jnp.float32),
    ],
)


@functools.partial(
    pl.kernel,
    out_type=jax.ShapeDtypeStruct((4 * 2 * B, D), jnp.float32),
    mesh=_mesh,
    scratch_types=[
        pltpu.MemorySpace.VMEM((2, 128), jnp.int32),
        pltpu.MemorySpace.VMEM((4, 128, D), jnp.float32),
        pltpu.SemaphoreType.DMA,
    ],
    compiler_params=_sc_params,
)
def _final_gather(t0, t1, t2, t3, idx_hbm, out_hbm, idxb, gbuf, sem):
    c = lax.axis_index("c")
    t = lax.axis_index("s")
    wid = t * NC + c
    pltpu.sync_copy(idx_hbm.at[wid], idxb)
    nrows = 2 * B
    for j in range(2):
        gd = [pltpu.async_copy(tref.at[idxb.at[j]], gbuf.at[tab], sem)
              for tab, tref in enumerate((t0, t1, t2, t3))]
        for dd in gd:
            dd.wait()
        base = wid * 256 + j * 128
        for tab in range(4):
            pltpu.sync_copy(gbuf.at[tab],
                            out_hbm.at[pl.ds(tab * nrows + base, 128)])


def kernel(user_emb, item_emb, edge_values, W_gc, b_gc, W_bi, b_bi,
           edge_index, users, items):
    ego0 = jnp.concatenate([user_emb, item_emb], axis=0)
    ego0_pad = jnp.pad(ego0, ((0, NPAD - NNODE), (0, 0)))
    ego_nm = ego0_pad.reshape(P2, 128)

    rc = jnp.pad(edge_index.astype(jnp.int32),
                 ((0, 0), (0, EP - E))).reshape(2, NS, NBLK, NSUB, 128)
    valp = jnp.pad(edge_values, (0, EP - E)).reshape(NS, NBLK, K)

    f32 = jnp.float32
    i2 = jnp.eye(2, dtype=f32)
    i4 = jnp.eye(4, dtype=f32)
    gg = jnp.kron(i2, jnp.ones((D, D), f32))
    h0 = jnp.concatenate([jnp.eye(DH, dtype=f32),
                          jnp.zeros((DH, DH), f32)], axis=1)
    h1 = jnp.concatenate([jnp.zeros((DH, DH), f32),
                          jnp.eye(DH, dtype=f32)], axis=1)
    pa = jnp.kron(i4, h0)
    pb = jnp.kron(i4, h1)

    norm_tabs = []
    for k in range(L):
        wg = jnp.kron(i2, W_gc[k])
        wb = jnp.kron(i2, W_bi[k])
        b2 = jnp.tile(b_gc[k] + b_bi[k], (1, 2))
        side_sp = _spmm(ego_nm.reshape(2 * NPAD, DH), rc, valp)
        ego_nm, norm_k = _tc_transform(side_sp.reshape(NC, P4, 128), ego_nm,
                                       pa, pb, wg, wb, b2, gg)
        norm_tabs.append(norm_k)

    idx_all = jnp.concatenate(
        [users.astype(jnp.int32), items.astype(jnp.int32) + N_USER]
    ).reshape(32, 2, 128)
    out4 = _final_gather(ego0_pad, norm_tabs[0].reshape(NPAD, D),
                         norm_tabs[1].reshape(NPAD, D),
                         norm_tabs[2].reshape(NPAD, D), idx_all)
    res = out4.reshape(4, 2 * B, D).transpose(1, 0, 2).reshape(2 * B, 4 * D)
    return res[:B], res[B:]


# speedup vs baseline: 1.1156x; 1.1156x over previous
import functools

import jax
import jax.numpy as jnp
from jax import lax
from jax.experimental import pallas as pl
from jax.experimental.pallas import tpu as pltpu
from jax.experimental.pallas import tpu_sc as plsc

N_USER = 25000
N_ITEM = 25000
NNODE = N_USER + N_ITEM
D = 64
DH = 32
L = 3
E = 800000
B = 4096

NC, NS = 2, 16
K = 256
NSUB = K // 128
NBLK = 196
ET = NBLK * K
EP = NS * ET
NPAD = 50176
RPT = NPAD // NS
P4 = NPAD // 4
P2 = NPAD // 2
RP = 1792
GRID = P4 // RP

_mesh = plsc.VectorSubcoreMesh(core_axis_name="c", subcore_axis_name="s")
_sc_params = pltpu.CompilerParams(use_tc_tiling_on_sc=False)


@functools.partial(
    pl.kernel,
    out_type=jax.ShapeDtypeStruct((NC, NPAD, DH), jnp.float32),
    mesh=_mesh,
    scratch_types=[
        pltpu.MemorySpace.VMEM_SHARED((NPAD, DH), jnp.float32),
        pltpu.MemorySpace.VMEM((4, NSUB, 128), jnp.int32),
        pltpu.MemorySpace.VMEM((4, NSUB, 128), jnp.int32),
        pltpu.MemorySpace.VMEM((4, K), jnp.float32),
        pltpu.MemorySpace.VMEM((2, NSUB, 128), jnp.int32),
        pltpu.MemorySpace.VMEM((2, K, DH), jnp.float32),
        pltpu.SemaphoreType.DMA((4,)),
        pltpu.SemaphoreType.DMA((2,)),
        pltpu.SemaphoreType.DMA((2,)),
    ],
    compiler_params=_sc_params,
)
def _spmm(tbl_hbm, rc_hbm, val_hbm, out_hbm,
          acc, colb, rowb, valb, rowS, gath, isem, gsem, ssem):
    c = lax.axis_index("c")
    t = lax.axis_index("s")

    def zrow(i, _):
        gath[0, i, pl.ds(0, 16)] = jnp.zeros((16,), jnp.float32)
        gath[0, i, pl.ds(16, 16)] = jnp.zeros((16,), jnp.float32)
        return 0
    lax.fori_loop(0, K, zrow, 0)
    zd = []
    for off in range(0, RPT, K):
        sz = min(K, RPT - off)
        zd.append(pltpu.async_copy(gath.at[0, pl.ds(0, sz)],
                                   acc.at[pl.ds(t * RPT + off, sz)],
                                   ssem.at[0]))
    for dd in zd:
        dd.wait()

    off_c = c

    def issue_idx(b, r):
        pltpu.async_copy(rc_hbm.at[1, t, b], colb.at[r], isem.at[r])
        pltpu.async_copy(rc_hbm.at[0, t, b], rowb.at[r], isem.at[r])
        pltpu.async_copy(val_hbm.at[t, b], valb.at[r], isem.at[r])

    def drain_idx(b, r):
        pltpu.make_async_copy(rc_hbm.at[1, t, b], colb.at[r],
                              isem.at[r]).wait()
        pltpu.make_async_copy(rc_hbm.at[0, t, b], rowb.at[r],
                              isem.at[r]).wait()
        pltpu.make_async_copy(val_hbm.at[t, b], valb.at[r],
                              isem.at[r]).wait()

    def drain_scat(s):
        for j in range(NSUB):
            pltpu.make_async_copy(gath.at[s, pl.ds(j * 128, 128)],
                                  acc.at[rowS.at[s, j]], ssem.at[s]).wait()

    def produce(b, r, s, first):
        drain_idx(b, r)

        def obody(ii, _):
            for j in range(NSUB):
                colb[r, j, pl.ds(ii * 16, 16)] = (
                    colb[r, j, pl.ds(ii * 16, 16)] * 2 + off_c)
            return 0
        lax.fori_loop(0, 8, obody, 0, unroll=True)

        def after_credit():
            for j in range(NSUB):
                def cbody(ii, _):
                    rowS[s, j, pl.ds(ii * 16, 16)] = rowb[r, j,
                                                          pl.ds(ii * 16, 16)]
                    return 0
                lax.fori_loop(0, 8, cbody, 0, unroll=True)

        if first:
            @pl.when(b >= 2)
            def _():
                drain_scat(s)
        else:
            drain_scat(s)
        after_credit()
        return [pltpu.async_copy(tbl_hbm.at[colb.at[r, j]],
                                 gath.at[s, pl.ds(j * 128, 128)], gsem.at[s])
                for j in range(NSUB)]

    def scale(s, r):
        @plsc.parallel_loop(0, K // 16, 1, unroll=2)
        def _(g):
            vv = valb[r, pl.ds(g * 16, 16)]
            base = g * 16
            for j in range(16):
                v = vv[j]
                gath[s, base + j, pl.ds(0, 16)] = (
                    gath[s, base + j, pl.ds(0, 16)] * v)
                gath[s, base + j, pl.ds(16, 16)] = (
                    gath[s, base + j, pl.ds(16, 16)] * v)

    def consume(x, r, s):
        pltpu.make_async_copy(tbl_hbm.at[colb.at[r, 0]],
                              gath.at[s, pl.ds(0, 128)], gsem.at[s]).wait()
        pltpu.make_async_copy(tbl_hbm.at[colb.at[r, 1]],
                              gath.at[s, pl.ds(128, 128)], gsem.at[s]).wait()
        scale(s, r)
        for j in range(NSUB):
            pltpu.async_copy(gath.at[s, pl.ds(j * 128, 128)],
                             acc.at[rowS.at[s, j]], ssem.at[s], add=True)

    for bb in range(3):
        issue_idx(bb, bb)
    plsc.subcore_barrier()

    NB = ET // K

    def quad_body(o, _):
        for u in range(4):
            b = 4 * o + u
            s = u % 2
            gd = produce(b, u, s, first=(u < 2))
            xu = (u - 1) % 4
            xs = (u - 1) % 2
            pref = b + 2
            if u == 0:
                @pl.when(o > 0)
                def _():
                    consume(b - 1, xu, xs)
                    issue_idx(pref, (u + 2) % 4)
            elif u == 1:
                consume(b - 1, xu, xs)
                issue_idx(pref, (u + 2) % 4)
            else:
                consume(b - 1, xu, xs)

                @pl.when(o < NB // 4 - 1)
                def _():
                    issue_idx(pref, (u + 2) % 4)
        return 0

    lax.fori_loop(0, NB // 4, quad_body, 0)
    consume(NB - 1, (NB - 1) % 4, (NB - 1) % 2)
    drain_scat(0)
    drain_scat(1)
    plsc.subcore_barrier()
    pltpu.sync_copy(acc.at[pl.ds(t * RPT, RPT)],
                    out_hbm.at[c, pl.ds(t * RPT, RPT)])


def _tc_transform_body(s_ref, e_ref, pa_ref, pb_ref, wg_ref, wb_ref,
                       b2_ref, gg_ref, ego_o_ref, norm_o_ref):
    f32 = jnp.float32
    snm = (jnp.dot(s_ref[0], pa_ref[...], preferred_element_type=f32)
           + jnp.dot(s_ref[1], pb_ref[...], preferred_element_type=f32)
           ).reshape(2 * RP, 128)
    e = e_ref[...]
    z = (jnp.dot(snm, wg_ref[...], preferred_element_type=f32)
         + jnp.dot(e * snm, wb_ref[...], preferred_element_type=f32)
         + b2_ref[...])
    y = jnp.where(z >= 0, z, 0.2 * z)
    nrmb = jnp.dot(y * y, gg_ref[...], preferred_element_type=f32)
    norm_o_ref[...] = y / jnp.maximum(jnp.sqrt(nrmb), 1e-12)
    ego_o_ref[...] = y


_tc_transform = pl.pallas_call(
    _tc_transform_body,
    grid=(GRID,),
    in_specs=[
        pl.BlockSpec((NC, RP, 128), lambda i: (0, i, 0)),
        pl.BlockSpec((2 * RP, 128), lambda i: (i, 0)),
        pl.BlockSpec((128, 256), lambda i: (0, 0)),
        pl.BlockSpec((128, 256), lambda i: (0, 0)),
        pl.BlockSpec((128, 128), lambda i: (0, 0)),
        pl.BlockSpec((128, 128), lambda i: (0, 0)),
        pl.BlockSpec((1, 128), lambda i: (0, 0)),
        pl.BlockSpec((128, 128), lambda i: (0, 0)),
    ],
    out_specs=[
        pl.BlockSpec((2 * RP, 128), lambda i: (i, 0)),
        pl.BlockSpec((2 * RP, 128), lambda i: (i, 0)),
    ],
    out_shape=[
        jax.ShapeDtypeStruct((P2, 128), jnp.float32),
        jax.ShapeDtypeStruct((P2, 128), jnp.float32),
    ],
)


@functools.partial(
    pl.kernel,
    out_type=jax.ShapeDtypeStruct((4 * 2 * B, D), jnp.float32),
    mesh=_mesh,
    scratch_types=[
        pltpu.MemorySpace.VMEM((2, 128), jnp.int32),
        pltpu.MemorySpace.VMEM((4, 128, D), jnp.float32),
        pltpu.SemaphoreType.DMA,
    ],
    compiler_params=_sc_params,
)
def _final_gather(t0, t1, t2, t3, idx_hbm, out_hbm, idxb, gbuf, sem):
    c = lax.axis_index("c")
    t = lax.axis_index("s")
    wid = t * NC + c
    pltpu.sync_copy(idx_hbm.at[wid], idxb)
    nrows = 2 * B
    for j in range(2):
        gd = [pltpu.async_copy(tref.at[idxb.at[j]], gbuf.at[tab], sem)
              for tab, tref in enumerate((t0, t1, t2, t3))]
        for dd in gd:
            dd.wait()
        base = wid * 256 + j * 128
        for tab in range(4):
            pltpu.sync_copy(gbuf.at[tab],
                            out_hbm.at[pl.ds(tab * nrows + base, 128)])


def kernel(user_emb, item_emb, edge_values, W_gc, b_gc, W_bi, b_bi,
           edge_index, users, items):
    ego0 = jnp.concatenate([user_emb, item_emb], axis=0)
    ego0_pad = jnp.pad(ego0, ((0, NPAD - NNODE), (0, 0)))
    ego_nm = ego0_pad.reshape(P2, 128)

    rc = jnp.pad(edge_index.astype(jnp.int32),
                 ((0, 0), (0, EP - E))).reshape(2, NS, NBLK, NSUB, 128)
    valp = jnp.pad(edge_values, (0, EP - E)).reshape(NS, NBLK, K)

    f32 = jnp.float32
    i2 = jnp.eye(2, dtype=f32)
    i4 = jnp.eye(4, dtype=f32)
    gg = jnp.kron(i2, jnp.ones((D, D), f32))
    h0 = jnp.concatenate([jnp.eye(DH, dtype=f32),
                          jnp.zeros((DH, DH), f32)], axis=1)
    h1 = jnp.concatenate([jnp.zeros((DH, DH), f32),
                          jnp.eye(DH, dtype=f32)], axis=1)
    pa = jnp.kron(i4, h0)
    pb = jnp.kron(i4, h1)

    norm_tabs = []
    for k in range(L):
        wg = jnp.kron(i2, W_gc[k])
        wb = jnp.kron(i2, W_bi[k])
        b2 = jnp.tile(b_gc[k] + b_bi[k], (1, 2))
        side_sp = _spmm(ego_nm.reshape(2 * NPAD, DH), rc, valp)
        ego_nm, norm_k = _tc_transform(side_sp.reshape(NC, P4, 128), ego_nm,
                                       pa, pb, wg, wb, b2, gg)
        norm_tabs.append(norm_k)

    idx_all = jnp.concatenate(
        [users.astype(jnp.int32), items.astype(jnp.int32) + N_USER]
    ).reshape(32, 2, 128)
    out4 = _final_gather(ego0_pad, norm_tabs[0].reshape(NPAD, D),
                         norm_tabs[1].reshape(NPAD, D),
                         norm_tabs[2].reshape(NPAD, D), idx_all)
    res = out4.reshape(4, 2 * B, D).transpose(1, 0, 2).reshape(2 * B, 4 * D)
    return res[:B], res[B:]
